# baseline (speedup 1.0000x reference)
import functools

import jax
import jax.numpy as jnp
from jax import lax
from jax.experimental import pallas as pl
from jax.experimental.pallas import tpu as pltpu, tpu_sc as plsc

BATCH = 4096
SEQ = 50
EMBED_DIM = 64
N = BATCH * SEQ

NUM_CORES = 2
NUM_SUBCORES = 16
NUM_WORKERS = NUM_CORES * NUM_SUBCORES
PER_WORKER = N // NUM_WORKERS
CHUNK = 128
NUM_CHUNKS = PER_WORKER // CHUNK
NUM_PAIRS = NUM_CHUNKS // 2
LANES = 16
NUM_BINS = 12

BB = 8


def _gather_kernel(iw_hbm, words_hbm, out_hbm,
                   idxw_v, wbuf0, wbuf1, obuf0, obuf1,
                   semw0, semw1, semo0, semo1):
    wid = lax.axis_index("s") * NUM_CORES + lax.axis_index("c")
    base = wid * PER_WORKER
    qbase = base // 2
    wbuf = (wbuf0, wbuf1)
    obuf = (obuf0, obuf1)
    semw = (semw0, semw1)
    semo = (semo0, semo1)

    pltpu.sync_copy(iw_hbm.at[pl.ds(wid * NUM_CHUNKS, NUM_CHUNKS), :], idxw_v)

    def gather_into(c, p):
        pltpu.async_copy(words_hbm.at[idxw_v.at[c]], wbuf[p], semw[p])

    gather_into(0, 0)
    gather_into(1, 1)

    def pair_body(i, carry):
        for p in range(2):
            c = i * 2 + p
            qoff = qbase + c * (CHUNK // 2)
            pltpu.make_async_copy(
                words_hbm.at[idxw_v.at[c]], wbuf[p], semw[p]).wait()

            @pl.when(i >= 1)
            def _wait_prev_scatter():
                pltpu.make_async_copy(
                    obuf[p], out_hbm.at[pl.ds(qoff, CHUNK // 2), :],
                    semo[p]).wait()

            @plsc.parallel_loop(0, CHUNK // 2, unroll=4)
            def _pack(q):
                for h in range(2):
                    for j in range(EMBED_DIM // LANES):
                        obuf[p][q, pl.ds(h * EMBED_DIM + j * LANES, LANES)] = (
                            wbuf[p][2 * q + h, pl.ds(j * LANES, LANES)])

            pltpu.async_copy(
                obuf[p], out_hbm.at[pl.ds(qoff, CHUNK // 2), :], semo[p])

            @pl.when(i < NUM_PAIRS - 1)
            def _prefetch():
                gather_into(c + 2, p)
        return carry

    lax.fori_loop(0, NUM_PAIRS, pair_body, 0)

    for p in range(2):
        qoff = qbase + (NUM_CHUNKS - 2 + p) * (CHUNK // 2)
        pltpu.make_async_copy(
            obuf[p], out_hbm.at[pl.ds(qoff, CHUNK // 2), :], semo[p]).wait()


def _sc_gather(iw, words_table):
    mesh = plsc.VectorSubcoreMesh(core_axis_name="c", subcore_axis_name="s")
    run = functools.partial(
        pl.kernel,
        mesh=mesh,
        compiler_params=pltpu.CompilerParams(use_tc_tiling_on_sc=False),
        out_type=jax.ShapeDtypeStruct((N // 2, 2 * EMBED_DIM), jnp.float32),
        scratch_types=[
            pltpu.VMEM((NUM_CHUNKS, CHUNK), jnp.int32),
            pltpu.VMEM((CHUNK, EMBED_DIM), jnp.float32),
            pltpu.VMEM((CHUNK, EMBED_DIM), jnp.float32),
            pltpu.VMEM((CHUNK // 2, 2 * EMBED_DIM), jnp.float32),
            pltpu.VMEM((CHUNK // 2, 2 * EMBED_DIM), jnp.float32),
            pltpu.SemaphoreType.DMA,
            pltpu.SemaphoreType.DMA,
            pltpu.SemaphoreType.DMA,
            pltpu.SemaphoreType.DMA,
        ],
    )(_gather_kernel)
    return run(iw, words_table)


def _add_kernel(x_ref, sids_ref, stab_ref, o_ref):
    x = x_ref[...]
    a = x[:, :EMBED_DIM]
    b = x[:, EMBED_DIM:]
    pairs = jnp.concatenate(
        [a[:, None, :], b[:, None, :]], axis=1)
    x2 = pairs.reshape(BB * SEQ, EMBED_DIM)
    sids3 = sids_ref[...].reshape(BB, SEQ, 1)
    oh3 = (sids3 == lax.broadcasted_iota(jnp.int32, (1, 1, NUM_BINS), 2)
           ).astype(jnp.float32)
    oh = oh3.reshape(BB * SEQ, NUM_BINS)
    emb = jnp.dot(oh, stab_ref[...], preferred_element_type=jnp.float32)
    o_ref[...] = (x2 + emb).reshape(BB, SEQ, EMBED_DIM)


def _tc_add(x, scores_ids, scores_table):
    grid = (BATCH // BB,)
    return pl.pallas_call(
        _add_kernel,
        grid=grid,
        in_specs=[
            pl.BlockSpec((BB * SEQ // 2, 2 * EMBED_DIM), lambda i: (i, 0)),
            pl.BlockSpec((BB, SEQ), lambda i: (i, 0)),
            pl.BlockSpec((NUM_BINS, EMBED_DIM), lambda i: (0, 0)),
        ],
        out_specs=pl.BlockSpec((BB, SEQ, EMBED_DIM), lambda i: (i, 0, 0)),
        out_shape=jax.ShapeDtypeStruct((BATCH, SEQ, EMBED_DIM), jnp.float32),
    )(x, scores_ids, scores_table)


@jax.jit
def kernel(input_ids, scores_ids, words_table, scores_table):
    iw = input_ids.reshape(N // CHUNK, CHUNK).astype(jnp.int32)
    isc = scores_ids.astype(jnp.int32)
    x = _sc_gather(iw, words_table)
    return _tc_add(x, isc, scores_table)


# speedup vs baseline: 1.9408x; 1.9408x over previous
import functools

import jax
import jax.numpy as jnp
from jax import lax
from jax.experimental import pallas as pl
from jax.experimental.pallas import tpu as pltpu, tpu_sc as plsc

BATCH = 4096
SEQ = 50
EMBED_DIM = 64
N = BATCH * SEQ

NUM_CORES = 2
NUM_SUBCORES = 16
NUM_WORKERS = NUM_CORES * NUM_SUBCORES
PER_WORKER = N // NUM_WORKERS
CHUNK = 128
NUM_CHUNKS = PER_WORKER // CHUNK
NUM_PAIRS = NUM_CHUNKS // 2
LANES = 16
NUM_BINS = 12


def _emb_sum_kernel(iw_hbm, is_hbm, words_hbm, scores_hbm, out_hbm,
                    idxw_v, idxs_v, stab,
                    wbuf0, wbuf1, obuf0, obuf1,
                    semw0, semw1, semo0, semo1):
    wid = lax.axis_index("s") * NUM_CORES + lax.axis_index("c")
    base = wid * PER_WORKER
    wbuf = (wbuf0, wbuf1)
    obuf = (obuf0, obuf1)
    semw = (semw0, semw1)
    semo = (semo0, semo1)

    pltpu.sync_copy(iw_hbm.at[pl.ds(base, PER_WORKER)], idxw_v)
    pltpu.sync_copy(is_hbm.at[pl.ds(base, PER_WORKER)], idxs_v)
    pltpu.sync_copy(scores_hbm, stab)

    def gather_into(c, p):
        off = c * CHUNK
        pltpu.async_copy(
            words_hbm.at[idxw_v.at[pl.ds(off, CHUNK)]], wbuf[p], semw[p])

    gather_into(0, 0)
    gather_into(1, 1)

    def pair_body(i, carry):
        for p in range(2):
            c = i * 2 + p
            off = c * CHUNK
            pltpu.make_async_copy(
                words_hbm.at[idxw_v.at[pl.ds(off, CHUNK)]],
                wbuf[p], semw[p]).wait()

            @pl.when(i >= 1)
            def _wait_prev_scatter():
                pltpu.make_async_copy(
                    obuf[p], out_hbm.at[pl.ds(base + off, CHUNK)],
                    semo[p]).wait()

            @plsc.parallel_loop(0, CHUNK, step=LANES)
            def _add_group(g):
                sidv = idxs_v[pl.ds(off + g, LANES)]
                for k in range(LANES):
                    sid = sidv[k]
                    for j in range(EMBED_DIM // LANES):
                        sl = pl.ds(j * LANES, LANES)
                        obuf[p][g + k, sl] = wbuf[p][g + k, sl] + stab[sid, sl]

            pltpu.async_copy(
                obuf[p], out_hbm.at[pl.ds(base + off, CHUNK)], semo[p])

            @pl.when(i < NUM_PAIRS - 1)
            def _prefetch():
                gather_into(c + 2, p)
        return carry

    lax.fori_loop(0, NUM_PAIRS, pair_body, 0)

    for p in range(2):
        off = (NUM_CHUNKS - 2 + p) * CHUNK
        pltpu.make_async_copy(
            obuf[p], out_hbm.at[pl.ds(base + off, CHUNK)], semo[p]).wait()


@jax.jit
def kernel(input_ids, scores_ids, words_table, scores_table):
    iw = input_ids.reshape(-1).astype(jnp.int32)
    isc = scores_ids.reshape(-1).astype(jnp.int32)
    mesh = plsc.VectorSubcoreMesh(core_axis_name="c", subcore_axis_name="s")
    run = functools.partial(
        pl.kernel,
        mesh=mesh,
        compiler_params=pltpu.CompilerParams(use_tc_tiling_on_sc=False),
        out_type=jax.ShapeDtypeStruct((N, EMBED_DIM), jnp.float32),
        scratch_types=[
            pltpu.VMEM((PER_WORKER,), jnp.int32),
            pltpu.VMEM((PER_WORKER,), jnp.int32),
            pltpu.VMEM((NUM_BINS, EMBED_DIM), jnp.float32),
            pltpu.VMEM((CHUNK, EMBED_DIM), jnp.float32),
            pltpu.VMEM((CHUNK, EMBED_DIM), jnp.float32),
            pltpu.VMEM((CHUNK, EMBED_DIM), jnp.float32),
            pltpu.VMEM((CHUNK, EMBED_DIM), jnp.float32),
            pltpu.SemaphoreType.DMA,
            pltpu.SemaphoreType.DMA,
            pltpu.SemaphoreType.DMA,
            pltpu.SemaphoreType.DMA,
        ],
    )(_emb_sum_kernel)
    out = run(iw, isc, words_table, scores_table)
    return out.reshape(BATCH, SEQ, EMBED_DIM)


# speedup vs baseline: 2.1412x; 1.1032x over previous
import functools

import jax
import jax.numpy as jnp
from jax import lax
from jax.experimental import pallas as pl
from jax.experimental.pallas import tpu as pltpu, tpu_sc as plsc

BATCH = 4096
SEQ = 50
EMBED_DIM = 64
N = BATCH * SEQ

NUM_CORES = 2
NUM_SUBCORES = 16
NUM_WORKERS = NUM_CORES * NUM_SUBCORES
PER_WORKER = N // NUM_WORKERS
CHUNK = 320
NUM_CHUNKS = PER_WORKER // CHUNK
NUM_PAIRS = NUM_CHUNKS // 2
LANES = 16
NUM_BINS = 12


def _emb_sum_kernel(iw_hbm, is_hbm, words_hbm, scores_hbm, out_hbm,
                    idxw_v, idxs_v, stab,
                    wbuf0, wbuf1, obuf0, obuf1,
                    semw0, semw1, semo0, semo1):
    wid = lax.axis_index("s") * NUM_CORES + lax.axis_index("c")
    base = wid * PER_WORKER
    wbuf = (wbuf0, wbuf1)
    obuf = (obuf0, obuf1)
    semw = (semw0, semw1)
    semo = (semo0, semo1)

    pltpu.sync_copy(iw_hbm.at[pl.ds(base, PER_WORKER)], idxw_v)
    pltpu.sync_copy(is_hbm.at[pl.ds(base, PER_WORKER)], idxs_v)
    pltpu.sync_copy(scores_hbm, stab)

    def gather_into(c, p):
        off = c * CHUNK
        pltpu.async_copy(
            words_hbm.at[idxw_v.at[pl.ds(off, CHUNK)]], wbuf[p], semw[p])

    gather_into(0, 0)
    gather_into(1, 1)

    def pair_body(i, carry):
        for p in range(2):
            c = i * 2 + p
            off = c * CHUNK
            pltpu.make_async_copy(
                words_hbm.at[idxw_v.at[pl.ds(off, CHUNK)]],
                wbuf[p], semw[p]).wait()

            @pl.when(i >= 1)
            def _wait_prev_scatter():
                pltpu.make_async_copy(
                    obuf[p], out_hbm.at[pl.ds(base + off, CHUNK)],
                    semo[p]).wait()

            @plsc.parallel_loop(0, CHUNK, step=LANES)
            def _add_group(g):
                sidv = idxs_v[pl.ds(off + g, LANES)]
                for k in range(LANES):
                    sid = sidv[k]
                    for j in range(EMBED_DIM // LANES):
                        sl = pl.ds(j * LANES, LANES)
                        obuf[p][g + k, sl] = wbuf[p][g + k, sl] + stab[sid, sl]

            pltpu.async_copy(
                obuf[p], out_hbm.at[pl.ds(base + off, CHUNK)], semo[p])

            @pl.when(i < NUM_PAIRS - 1)
            def _prefetch():
                gather_into(c + 2, p)
        return carry

    lax.fori_loop(0, NUM_PAIRS, pair_body, 0)

    for p in range(2):
        off = (NUM_CHUNKS - 2 + p) * CHUNK
        pltpu.make_async_copy(
            obuf[p], out_hbm.at[pl.ds(base + off, CHUNK)], semo[p]).wait()


@jax.jit
def kernel(input_ids, scores_ids, words_table, scores_table):
    iw = input_ids.reshape(-1).astype(jnp.int32)
    isc = scores_ids.reshape(-1).astype(jnp.int32)
    mesh = plsc.VectorSubcoreMesh(core_axis_name="c", subcore_axis_name="s")
    run = functools.partial(
        pl.kernel,
        mesh=mesh,
        compiler_params=pltpu.CompilerParams(use_tc_tiling_on_sc=False),
        out_type=jax.ShapeDtypeStruct((N, EMBED_DIM), jnp.float32),
        scratch_types=[
            pltpu.VMEM((PER_WORKER,), jnp.int32),
            pltpu.VMEM((PER_WORKER,), jnp.int32),
            pltpu.VMEM((NUM_BINS, EMBED_DIM), jnp.float32),
            pltpu.VMEM((CHUNK, EMBED_DIM), jnp.float32),
            pltpu.VMEM((CHUNK, EMBED_DIM), jnp.float32),
            pltpu.VMEM((CHUNK, EMBED_DIM), jnp.float32),
            pltpu.VMEM((CHUNK, EMBED_DIM), jnp.float32),
            pltpu.SemaphoreType.DMA,
            pltpu.SemaphoreType.DMA,
            pltpu.SemaphoreType.DMA,
            pltpu.SemaphoreType.DMA,
        ],
    )(_emb_sum_kernel)
    out = run(iw, isc, words_table, scores_table)
    return out.reshape(BATCH, SEQ, EMBED_DIM)


# speedup vs baseline: 2.1707x; 1.0138x over previous
import functools

import jax
import jax.numpy as jnp
from jax import lax
from jax.experimental import pallas as pl
from jax.experimental.pallas import tpu as pltpu, tpu_sc as plsc

BATCH = 4096
SEQ = 50
EMBED_DIM = 64
N = BATCH * SEQ

NUM_CORES = 2
NUM_SUBCORES = 16
NUM_WORKERS = NUM_CORES * NUM_SUBCORES
PER_WORKER = N // NUM_WORKERS
CHUNK = 400
NUM_CHUNKS = PER_WORKER // CHUNK
NUM_PAIRS = NUM_CHUNKS // 2
LANES = 16
NUM_BINS = 12


def _emb_sum_kernel(iw_hbm, is_hbm, words_hbm, scores_hbm, out_hbm,
                    idxw_v, idxs_v, stab,
                    wbuf0, wbuf1, obuf0, obuf1,
                    semw0, semw1, semo0, semo1):
    wid = lax.axis_index("s") * NUM_CORES + lax.axis_index("c")
    base = wid * PER_WORKER
    wbuf = (wbuf0, wbuf1)
    obuf = (obuf0, obuf1)
    semw = (semw0, semw1)
    semo = (semo0, semo1)

    pltpu.sync_copy(iw_hbm.at[pl.ds(base, PER_WORKER)], idxw_v)
    pltpu.sync_copy(is_hbm.at[pl.ds(base, PER_WORKER)], idxs_v)
    pltpu.sync_copy(scores_hbm, stab)

    def gather_into(c, p):
        off = c * CHUNK
        pltpu.async_copy(
            words_hbm.at[idxw_v.at[pl.ds(off, CHUNK)]], wbuf[p], semw[p])

    gather_into(0, 0)
    gather_into(1, 1)

    def pair_body(i, carry):
        for p in range(2):
            c = i * 2 + p
            off = c * CHUNK
            pltpu.make_async_copy(
                words_hbm.at[idxw_v.at[pl.ds(off, CHUNK)]],
                wbuf[p], semw[p]).wait()

            @pl.when(i >= 1)
            def _wait_prev_scatter():
                pltpu.make_async_copy(
                    obuf[p], out_hbm.at[pl.ds(base + off, CHUNK)],
                    semo[p]).wait()

            @plsc.parallel_loop(0, CHUNK, step=LANES)
            def _add_group(g):
                sidv = idxs_v[pl.ds(off + g, LANES)]
                for k in range(LANES):
                    sid = sidv[k]
                    for j in range(EMBED_DIM // LANES):
                        sl = pl.ds(j * LANES, LANES)
                        obuf[p][g + k, sl] = wbuf[p][g + k, sl] + stab[sid, sl]

            pltpu.async_copy(
                obuf[p], out_hbm.at[pl.ds(base + off, CHUNK)], semo[p])

            @pl.when(i < NUM_PAIRS - 1)
            def _prefetch():
                gather_into(c + 2, p)
        return carry

    lax.fori_loop(0, NUM_PAIRS, pair_body, 0)

    for p in range(2):
        off = (NUM_CHUNKS - 2 + p) * CHUNK
        pltpu.make_async_copy(
            obuf[p], out_hbm.at[pl.ds(base + off, CHUNK)], semo[p]).wait()


@jax.jit
def kernel(input_ids, scores_ids, words_table, scores_table):
    iw = input_ids.reshape(-1).astype(jnp.int32)
    isc = scores_ids.reshape(-1).astype(jnp.int32)
    mesh = plsc.VectorSubcoreMesh(core_axis_name="c", subcore_axis_name="s")
    run = functools.partial(
        pl.kernel,
        mesh=mesh,
        compiler_params=pltpu.CompilerParams(use_tc_tiling_on_sc=False),
        out_type=jax.ShapeDtypeStruct((N, EMBED_DIM), jnp.float32),
        scratch_types=[
            pltpu.VMEM((PER_WORKER,), jnp.int32),
            pltpu.VMEM((PER_WORKER,), jnp.int32),
            pltpu.VMEM((NUM_BINS, EMBED_DIM), jnp.float32),
            pltpu.VMEM((CHUNK, EMBED_DIM), jnp.float32),
            pltpu.VMEM((CHUNK, EMBED_DIM), jnp.float32),
            pltpu.VMEM((CHUNK, EMBED_DIM), jnp.float32),
            pltpu.VMEM((CHUNK, EMBED_DIM), jnp.float32),
            pltpu.SemaphoreType.DMA,
            pltpu.SemaphoreType.DMA,
            pltpu.SemaphoreType.DMA,
            pltpu.SemaphoreType.DMA,
        ],
    )(_emb_sum_kernel)
    out = run(iw, isc, words_table, scores_table)
    return out.reshape(BATCH, SEQ, EMBED_DIM)


# speedup vs baseline: 2.1763x; 1.0026x over previous
import functools

import jax
import jax.numpy as jnp
from jax import lax
from jax.experimental import pallas as pl
from jax.experimental.pallas import tpu as pltpu, tpu_sc as plsc

BATCH = 4096
SEQ = 50
EMBED_DIM = 64
N = BATCH * SEQ

NUM_CORES = 2
NUM_SUBCORES = 16
NUM_WORKERS = NUM_CORES * NUM_SUBCORES
PER_WORKER = N // NUM_WORKERS
CHUNK = 400
NUM_CHUNKS = PER_WORKER // CHUNK
NUM_PAIRS = NUM_CHUNKS // 2
LANES = 16
NUM_BINS = 12


def _emb_sum_kernel(iw_hbm, is_hbm, words_hbm, scores_hbm, out_hbm,
                    idxw_v, idxs_v, stab,
                    wbuf0, wbuf1, obuf0, obuf1,
                    semw0, semw1, semo0, semo1):
    wid = lax.axis_index("s") * NUM_CORES + lax.axis_index("c")
    base = wid * PER_WORKER
    wbuf = (wbuf0, wbuf1)
    obuf = (obuf0, obuf1)
    semw = (semw0, semw1)
    semo = (semo0, semo1)

    pltpu.sync_copy(iw_hbm.at[pl.ds(base, PER_WORKER)], idxw_v)
    pltpu.sync_copy(is_hbm.at[pl.ds(base, PER_WORKER)], idxs_v)
    pltpu.sync_copy(scores_hbm, stab)

    def gather_into(c, p):
        off = c * CHUNK
        pltpu.async_copy(
            words_hbm.at[idxw_v.at[pl.ds(off, CHUNK)]], wbuf[p], semw[p])

    gather_into(0, 0)
    gather_into(1, 1)

    def pair_body(i, carry):
        for p in range(2):
            c = i * 2 + p
            off = c * CHUNK
            pltpu.make_async_copy(
                words_hbm.at[idxw_v.at[pl.ds(off, CHUNK)]],
                wbuf[p], semw[p]).wait()

            @pl.when(i >= 1)
            def _wait_prev_scatter():
                pltpu.make_async_copy(
                    obuf[p], out_hbm.at[pl.ds((base + off) // 2, CHUNK // 2)],
                    semo[p]).wait()

            @plsc.parallel_loop(0, CHUNK, step=LANES)
            def _add_group(g):
                sidv = idxs_v[pl.ds(off + g, LANES)]
                for k in range(LANES):
                    sid = sidv[k]
                    q = (g + k) // 2
                    hoff = ((g + k) % 2) * EMBED_DIM
                    for j in range(EMBED_DIM // LANES):
                        sl = pl.ds(j * LANES, LANES)
                        obuf[p][q, pl.ds(hoff + j * LANES, LANES)] = (
                            wbuf[p][g + k, sl] + stab[sid, sl])

            pltpu.async_copy(
                obuf[p], out_hbm.at[pl.ds((base + off) // 2, CHUNK // 2)],
                semo[p])

            @pl.when(i < NUM_PAIRS - 1)
            def _prefetch():
                gather_into(c + 2, p)
        return carry

    lax.fori_loop(0, NUM_PAIRS, pair_body, 0)

    for p in range(2):
        off = (NUM_CHUNKS - 2 + p) * CHUNK
        pltpu.make_async_copy(
            obuf[p], out_hbm.at[pl.ds((base + off) // 2, CHUNK // 2)],
            semo[p]).wait()


@jax.jit
def kernel(input_ids, scores_ids, words_table, scores_table):
    iw = input_ids.reshape(-1).astype(jnp.int32)
    isc = scores_ids.reshape(-1).astype(jnp.int32)
    mesh = plsc.VectorSubcoreMesh(core_axis_name="c", subcore_axis_name="s")
    run = functools.partial(
        pl.kernel,
        mesh=mesh,
        compiler_params=pltpu.CompilerParams(use_tc_tiling_on_sc=False),
        out_type=jax.ShapeDtypeStruct((N // 2, 2 * EMBED_DIM), jnp.float32),
        scratch_types=[
            pltpu.VMEM((PER_WORKER,), jnp.int32),
            pltpu.VMEM((PER_WORKER,), jnp.int32),
            pltpu.VMEM((NUM_BINS, EMBED_DIM), jnp.float32),
            pltpu.VMEM((CHUNK, EMBED_DIM), jnp.float32),
            pltpu.VMEM((CHUNK, EMBED_DIM), jnp.float32),
            pltpu.VMEM((CHUNK // 2, 2 * EMBED_DIM), jnp.float32),
            pltpu.VMEM((CHUNK // 2, 2 * EMBED_DIM), jnp.float32),
            pltpu.SemaphoreType.DMA,
            pltpu.SemaphoreType.DMA,
            pltpu.SemaphoreType.DMA,
            pltpu.SemaphoreType.DMA,
        ],
    )(_emb_sum_kernel)
    out = run(iw, isc, words_table, scores_table)
    return out.reshape(BATCH, SEQ, EMBED_DIM)


# speedup vs baseline: 2.1765x; 1.0001x over previous
import functools

import jax
import jax.numpy as jnp
from jax import lax
from jax.experimental import pallas as pl
from jax.experimental.pallas import tpu as pltpu, tpu_sc as plsc

BATCH = 4096
SEQ = 50
EMBED_DIM = 64
N = BATCH * SEQ

NUM_CORES = 2
NUM_SUBCORES = 16
NUM_WORKERS = NUM_CORES * NUM_SUBCORES
PER_WORKER = N // NUM_WORKERS
CHUNK = 400
NUM_CHUNKS = PER_WORKER // CHUNK
NUM_PAIRS = NUM_CHUNKS // 2
LANES = 16
NUM_BINS = 12


def _emb_sum_kernel(iw_hbm, is_hbm, words_hbm, scores_hbm, out_hbm,
                    idxw_v, idxs_v, stab,
                    wbuf0, wbuf1, obuf0, obuf1,
                    semw0, semw1, semo0, semo1):
    wid = lax.axis_index("s") * NUM_CORES + lax.axis_index("c")
    base = wid * PER_WORKER
    wbuf = (wbuf0, wbuf1)
    obuf = (obuf0, obuf1)
    semw = (semw0, semw1)
    semo = (semo0, semo1)

    pltpu.sync_copy(iw_hbm.at[pl.ds(base, PER_WORKER)], idxw_v)
    pltpu.sync_copy(is_hbm.at[pl.ds(base, PER_WORKER)], idxs_v)
    pltpu.sync_copy(scores_hbm, stab)

    def gather_into(c, p):
        off = c * CHUNK
        pltpu.async_copy(
            words_hbm.at[idxw_v.at[pl.ds(off, CHUNK)]], wbuf[p], semw[p])

    gather_into(0, 0)
    gather_into(1, 1)

    def pair_body(i, carry):
        for p in range(2):
            c = i * 2 + p
            off = c * CHUNK
            pltpu.make_async_copy(
                words_hbm.at[idxw_v.at[pl.ds(off, CHUNK)]],
                wbuf[p], semw[p]).wait()

            @pl.when(i >= 1)
            def _wait_prev_scatter():
                pltpu.make_async_copy(
                    obuf[p], out_hbm.at[pl.ds(base + off, CHUNK)],
                    semo[p]).wait()

            @plsc.parallel_loop(0, CHUNK, step=LANES)
            def _add_group(g):
                sidv = idxs_v[pl.ds(off + g, LANES)]
                for k in range(LANES):
                    sid = sidv[k]
                    for j in range(EMBED_DIM // LANES):
                        sl = pl.ds(j * LANES, LANES)
                        obuf[p][g + k, sl] = wbuf[p][g + k, sl] + stab[sid, sl]

            pltpu.async_copy(
                obuf[p], out_hbm.at[pl.ds(base + off, CHUNK)], semo[p])

            @pl.when(i < NUM_PAIRS - 1)
            def _prefetch():
                gather_into(c + 2, p)
        return carry

    lax.fori_loop(0, NUM_PAIRS, pair_body, 0)

    for p in range(2):
        off = (NUM_CHUNKS - 2 + p) * CHUNK
        pltpu.make_async_copy(
            obuf[p], out_hbm.at[pl.ds(base + off, CHUNK)], semo[p]).wait()


@jax.jit
def kernel(input_ids, scores_ids, words_table, scores_table):
    iw = input_ids.reshape(-1).astype(jnp.int32)
    isc = scores_ids.reshape(-1).astype(jnp.int32)
    mesh = plsc.VectorSubcoreMesh(core_axis_name="c", subcore_axis_name="s")
    run = functools.partial(
        pl.kernel,
        mesh=mesh,
        compiler_params=pltpu.CompilerParams(use_tc_tiling_on_sc=False),
        out_type=jax.ShapeDtypeStruct((N, EMBED_DIM), jnp.float32),
        scratch_types=[
            pltpu.VMEM((PER_WORKER,), jnp.int32),
            pltpu.VMEM((PER_WORKER,), jnp.int32),
            pltpu.VMEM((NUM_BINS, EMBED_DIM), jnp.float32),
            pltpu.VMEM((CHUNK, EMBED_DIM), jnp.float32),
            pltpu.VMEM((CHUNK, EMBED_DIM), jnp.float32),
            pltpu.VMEM((CHUNK, EMBED_DIM), jnp.float32),
            pltpu.VMEM((CHUNK, EMBED_DIM), jnp.float32),
            pltpu.SemaphoreType.DMA,
            pltpu.SemaphoreType.DMA,
            pltpu.SemaphoreType.DMA,
            pltpu.SemaphoreType.DMA,
        ],
    )(_emb_sum_kernel)
    out = run(iw, isc, words_table, scores_table)
    return out.reshape(BATCH, SEQ, EMBED_DIM)
